# baseline (speedup 1.0000x reference)
import jax
import jax.numpy as jnp
from jax import lax
from jax.experimental import pallas as pl

_LR, _B1, _B2, _EPS = 1e-3, 0.9, 0.999, 1e-8
_STEPS = 10
_K = 3


def _rownorm(x):
    return jnp.maximum(jnp.sqrt(jnp.sum(x * x, axis=1, keepdims=True)), 1e-12)


def _train_body(sup_ref, lab_col_ref, w_ref, dsup_ref, s2_ref):
    sup = sup_ref[...]
    n, d = sup.shape
    sn = sup / _rownorm(sup)

    lc = lab_col_ref[...]
    ncls = 64
    onehot = jnp.where(
        lc == lax.broadcasted_iota(jnp.int32, (n, ncls), 1), 1.0, 0.0
    ).astype(jnp.float32)
    counts = jnp.sum(onehot, axis=0, keepdims=True)
    s_all = jnp.float32(n) * jnp.float32(n) - 2.0 * jnp.sum(counts * counts)
    inv_ssum = 2.0 / (s_all + jnp.float32(n))

    wr = lax.broadcasted_iota(jnp.int32, (d, d), 0)
    wc = lax.broadcasted_iota(jnp.int32, (d, d), 1)
    eye = jnp.where(wr == wc, 1.0, 0.0).astype(jnp.float32)
    w = eye
    mom = jnp.zeros((d, d), jnp.float32)
    vel = jnp.zeros((d, d), jnp.float32)
    sn_bf = sn.astype(jnp.bfloat16)

    for i in range(_STEPS):
        dw_bf = (w - eye).astype(jnp.bfloat16)
        z = sn + lax.dot_general(sn_bf, dw_bf, (((1,), (1,)), ((), ())),
                                 preferred_element_type=jnp.float32)
        zr = _rownorm(z)
        t = z / zr
        csum = lax.dot_general(onehot, t, (((0,), (0,)), ((), ())),
                               preferred_element_type=jnp.float32)
        persum = jnp.dot(onehot, csum, preferred_element_type=jnp.float32)
        g = (jnp.sum(t, axis=0, keepdims=True) - 2.0 * persum + t) * inv_ssum
        dz = (g - t * jnp.sum(t * g, axis=1, keepdims=True)) / zr
        gw = lax.dot_general(dz, sn, (((0,), (0,)), ((), ())),
                             preferred_element_type=jnp.float32)
        mom = _B1 * mom + (1.0 - _B1) * gw
        vel = _B2 * vel + (1.0 - _B2) * gw * gw
        mhat = mom / (1.0 - _B1 ** (i + 1))
        vhat = vel / (1.0 - _B2 ** (i + 1))
        w = w - _LR * mhat / (jnp.sqrt(vhat) + _EPS)

    dsup = lax.dot_general(sn, w, (((1,), (1,)), ((), ())),
                           preferred_element_type=jnp.float32)
    w_ref[...] = w
    dsup_ref[...] = dsup
    s2_ref[...] = jnp.sum(dsup * dsup, axis=1, keepdims=True)


def _knn_body(q_ref, w_ref, dsup_ref, s2_ref, out_ref):
    q = q_ref[...]
    qn = q / _rownorm(q)
    dq = lax.dot_general(qn, w_ref[...], (((1,), (1,)), ((), ())),
                         preferred_element_type=jnp.float32)
    ds = dsup_ref[...]
    cross = lax.dot_general(dq, ds, (((1,), (1,)), ((), ())),
                            preferred_element_type=jnp.float32)
    q2 = jnp.sum(dq * dq, axis=1, keepdims=True)
    d2 = jnp.maximum(q2 + s2_ref[...] - 2.0 * cross, 0.0)

    nsup = d2.shape[1]
    iota = lax.broadcasted_iota(jnp.int32, d2.shape, 1)
    acc = jnp.zeros((d2.shape[0], 1), jnp.float32)
    for _ in range(_K):
        mv = jnp.min(d2, axis=1, keepdims=True)
        acc = acc + jnp.sqrt(mv)
        idx = jnp.min(jnp.where(d2 == mv, iota, nsup), axis=1, keepdims=True)
        d2 = jnp.where(iota == idx, jnp.inf, d2)
    out_ref[...] = 1.0 - acc * (1.0 / _K)


def kernel(support_features, support_labels, query_features, query_labels):
    n, d = support_features.shape
    nq = query_features.shape[0]
    lab_col = support_labels.astype(jnp.int32).reshape(n, 1)

    w, dsup, s2 = pl.pallas_call(
        _train_body,
        out_shape=[
            jax.ShapeDtypeStruct((d, d), jnp.float32),
            jax.ShapeDtypeStruct((n, d), jnp.float32),
            jax.ShapeDtypeStruct((n, 1), jnp.float32),
        ],
    )(support_features, lab_col)

    s2_row = s2.reshape(1, n)
    bq = 1024
    grid = nq // bq
    out = pl.pallas_call(
        _knn_body,
        grid=(grid,),
        in_specs=[
            pl.BlockSpec((bq, d), lambda i: (i, 0)),
            pl.BlockSpec((d, d), lambda i: (0, 0)),
            pl.BlockSpec((n, d), lambda i: (0, 0)),
            pl.BlockSpec((1, n), lambda i: (0, 0)),
        ],
        out_specs=pl.BlockSpec((bq, 1), lambda i: (i, 0)),
        out_shape=jax.ShapeDtypeStruct((nq, 1), jnp.float32),
    )(query_features, w, dsup, s2_row)
    return out.reshape(nq)


# speedup vs baseline: 1.0598x; 1.0598x over previous
import jax
import jax.numpy as jnp
from jax import lax
from jax.experimental import pallas as pl

_LR, _B1, _B2, _EPS = 1e-3, 0.9, 0.999, 1e-8
_STEPS = 10
_K = 3


def _rownorm(x):
    return jnp.maximum(jnp.sqrt(jnp.sum(x * x, axis=1, keepdims=True)), 1e-12)


def _train_body(sup_ref, lab_col_ref, w_ref, dsup_ref, s2_ref):
    sup = sup_ref[...]
    n, d = sup.shape
    sn = sup / _rownorm(sup)

    lc = lab_col_ref[...]
    ncls = 64
    onehot = jnp.where(
        lc == lax.broadcasted_iota(jnp.int32, (n, ncls), 1), 1.0, 0.0
    ).astype(jnp.float32)
    counts = jnp.sum(onehot, axis=0, keepdims=True)
    s_all = jnp.float32(n) * jnp.float32(n) - 2.0 * jnp.sum(counts * counts)
    inv_ssum = 2.0 / (s_all + jnp.float32(n))

    wr = lax.broadcasted_iota(jnp.int32, (d, d), 0)
    wc = lax.broadcasted_iota(jnp.int32, (d, d), 1)
    w = jnp.where(wr == wc, 1.0, 0.0).astype(jnp.float32)
    mom = jnp.zeros((d, d), jnp.float32)
    vel = jnp.zeros((d, d), jnp.float32)

    for i in range(_STEPS):
        z = lax.dot_general(sn, w, (((1,), (1,)), ((), ())),
                            preferred_element_type=jnp.float32)
        izr = lax.rsqrt(jnp.sum(z * z, axis=1, keepdims=True))
        t = z * izr
        csum = lax.dot_general(onehot, t, (((0,), (0,)), ((), ())),
                               preferred_element_type=jnp.float32)
        persum = jnp.dot(onehot, csum, preferred_element_type=jnp.float32)
        colsum = jnp.sum(csum, axis=0, keepdims=True)
        g = (colsum - 2.0 * persum + t) * inv_ssum
        dz = (g - t * jnp.sum(t * g, axis=1, keepdims=True)) * izr
        gw = lax.dot_general(dz, sn, (((0,), (0,)), ((), ())),
                             preferred_element_type=jnp.float32)
        mom = _B1 * mom + (1.0 - _B1) * gw
        vel = _B2 * vel + (1.0 - _B2) * gw * gw
        c1i = 1.0 / (1.0 - _B1 ** (i + 1))
        c2i = 1.0 / (1.0 - _B2 ** (i + 1))
        w = w - (_LR * c1i) * mom / (jnp.sqrt(vel * c2i) + _EPS)

    dsup = lax.dot_general(sn, w, (((1,), (1,)), ((), ())),
                           preferred_element_type=jnp.float32)
    w_ref[...] = w
    dsup_ref[...] = dsup
    s2_ref[...] = jnp.sum(dsup * dsup, axis=1, keepdims=True)


def _knn_body(q_ref, w_ref, dsup_ref, s2_ref, out_ref):
    q = q_ref[...]
    qn = q / _rownorm(q)
    dq = lax.dot_general(qn, w_ref[...], (((1,), (1,)), ((), ())),
                         preferred_element_type=jnp.float32)
    ds = dsup_ref[...]
    cross = lax.dot_general(dq, ds, (((1,), (1,)), ((), ())),
                            preferred_element_type=jnp.float32)
    q2 = jnp.sum(dq * dq, axis=1, keepdims=True)
    d2 = jnp.maximum(q2 + s2_ref[...] - 2.0 * cross, 0.0)

    inf = jnp.float32(jnp.inf)
    m1 = jnp.min(d2, axis=1, keepdims=True)
    c1 = jnp.sum(jnp.where(d2 <= m1, 1.0, 0.0), axis=1, keepdims=True)
    masked = jnp.where(d2 > m1, d2, inf)
    m2 = jnp.min(masked, axis=1, keepdims=True)
    c2 = jnp.sum(jnp.where(masked <= m2, 1.0, 0.0), axis=1, keepdims=True)
    masked2 = jnp.where(masked > m2, masked, inf)
    m3 = jnp.min(masked2, axis=1, keepdims=True)

    k1 = jnp.minimum(c1, 3.0)
    k2 = jnp.minimum(c2, 3.0 - k1)
    k3 = 3.0 - k1 - k2
    s1 = jnp.sqrt(m1)
    s2v = jnp.sqrt(jnp.where(m2 < inf, m2, 0.0))
    s3v = jnp.sqrt(jnp.where(m3 < inf, m3, 0.0))
    sumd = k1 * s1 + k2 * s2v + k3 * s3v
    out_ref[...] = 1.0 - sumd * (1.0 / _K)


def kernel(support_features, support_labels, query_features, query_labels):
    n, d = support_features.shape
    nq = query_features.shape[0]
    lab_col = support_labels.astype(jnp.int32).reshape(n, 1)

    w, dsup, s2 = pl.pallas_call(
        _train_body,
        out_shape=[
            jax.ShapeDtypeStruct((d, d), jnp.float32),
            jax.ShapeDtypeStruct((n, d), jnp.float32),
            jax.ShapeDtypeStruct((n, 1), jnp.float32),
        ],
    )(support_features, lab_col)

    s2_row = s2.reshape(1, n)
    bq = 1024
    grid = nq // bq
    out = pl.pallas_call(
        _knn_body,
        grid=(grid,),
        in_specs=[
            pl.BlockSpec((bq, d), lambda i: (i, 0)),
            pl.BlockSpec((d, d), lambda i: (0, 0)),
            pl.BlockSpec((n, d), lambda i: (0, 0)),
            pl.BlockSpec((1, n), lambda i: (0, 0)),
        ],
        out_specs=pl.BlockSpec((bq, 1), lambda i: (i, 0)),
        out_shape=jax.ShapeDtypeStruct((nq, 1), jnp.float32),
    )(query_features, w, dsup, s2_row)
    return out.reshape(nq)
